# baseline (speedup 1.0000x reference)
import functools

import jax
import jax.numpy as jnp
from jax import lax
from jax.experimental import pallas as pl
from jax.experimental.pallas import tpu as pltpu
from jax.experimental.pallas import tpu_sc as plsc


_LANES = 128
_ROWS = 2
_TB = 8
_B_TC = 6272


def _sc_gather(x, tab_p):
    B, S = x.shape
    V = tab_p.shape[0]
    idx = x.reshape(B // _ROWS, _ROWS, S)
    w0 = _LANES
    w1 = S - _LANES

    mesh = plsc.VectorSubcoreMesh(core_axis_name="core",
                                  subcore_axis_name="subcore")

    @functools.partial(
        pl.kernel,
        out_type=jax.ShapeDtypeStruct((B, S, _LANES), tab_p.dtype),
        mesh=mesh,
        scratch_types=[
            pltpu.VMEM_SHARED((V, _LANES), jnp.float32),
            pltpu.SemaphoreType.DMA,
        ],
    )
    def gather_kernel(tab_hbm, i_hbm, o_hbm, tab_shared, sem):
        sid = lax.axis_index("subcore")

        @pl.when(sid == 0)
        def _stage_table():
            pltpu.sync_copy(tab_hbm, tab_shared)

        plsc.subcore_barrier()

        def body(i_vmem, o_vmem):
            copies = []
            for r in range(_ROWS):
                copies.append(pltpu.async_copy(
                    tab_shared.at[i_vmem.at[0, r, pl.ds(0, w0)]],
                    o_vmem.at[r, pl.ds(0, w0)], sem))
                copies.append(pltpu.async_copy(
                    tab_shared.at[i_vmem.at[0, r, pl.ds(w0, w1)]],
                    o_vmem.at[r, pl.ds(w0, w1)], sem))
            for c in copies:
                c.wait()

        pltpu.emit_pipeline(
            body,
            grid=(B // _ROWS,),
            in_specs=[pl.BlockSpec((1, _ROWS, S),
                                   index_map=lambda i: (i, 0, 0))],
            out_specs=[pl.BlockSpec((_ROWS, S, _LANES),
                                    index_map=lambda i: (i, 0, 0))],
            core_axis_name=("core", "subcore"),
            dimension_semantics=(pltpu.PARALLEL,),
        )(i_hbm, o_hbm)

    return gather_kernel(tab_p, idx)


def _tc_onehot(x, tab_bf, E):
    B, S = x.shape
    V = tab_bf.shape[0]

    def body(idx_ref, tab_ref, o_ref):
        idx3 = idx_ref[...][:, :, None]
        iota3 = lax.broadcasted_iota(jnp.int32, (_TB, S, V), 2)
        oh = (idx3 == iota3).astype(jnp.bfloat16)
        o_ref[...] = lax.dot_general(
            oh, tab_ref[...], (((2,), (0,)), ((), ())),
            preferred_element_type=jnp.float32)

    return pl.pallas_call(
        body,
        out_shape=jax.ShapeDtypeStruct((B, S, E), jnp.float32),
        grid=(B // _TB,),
        in_specs=[pl.BlockSpec((_TB, S), lambda i: (i, 0)),
                  pl.BlockSpec((V, E), lambda i: (0, 0))],
        out_specs=pl.BlockSpec((_TB, S, E), lambda i: (i, 0, 0)),
    )(x, tab_bf)


def kernel(x, table):
    B, S = x.shape
    V, E = table.shape
    b_sc = B - _B_TC
    tab_p = jnp.pad(table, ((0, 0), (0, _LANES - E)))
    tab_bf = table.astype(jnp.bfloat16)
    sc_out = _sc_gather(x[:b_sc], tab_p)[:, :, :E]
    tc_out = _tc_onehot(x[b_sc:], tab_bf, E)
    return jnp.concatenate([sc_out, tc_out], axis=0)
